# Initial kernel scaffold; baseline (speedup 1.0000x reference)
import jax
import jax.numpy as jnp
from jax.experimental import pallas as pl


def kernel(x, r_weight, l_weight):
    raise NotImplementedError("write your pallas kernel here")


# speedup vs baseline: 66.4552x; 66.4552x over previous
import functools

import jax
import jax.numpy as jnp
from jax import lax
from jax.experimental import pallas as pl
from jax.experimental.pallas import tpu as pltpu
from jax.experimental.pallas import tpu_sc as plsc

IN_F = 128
OUT_F = 64
P = 1000
EPS = 1e-06
BATCH = 16384

_NC = 2
_NS = 16
_NW = _NC * _NS
_BPW = BATCH // _NW
_SB = 16
_NSB = _BPW // _SB
_TW = 2 * OUT_F


def _prep_body(x_ref, xs_ref, idx_ref):
    x = x_ref[...]
    mins = jnp.min(x, axis=0, keepdims=True)
    maxs = jnp.max(x, axis=0, keepdims=True)
    xs = (x - mins) / (maxs - mins + EPS) * (P - 1)
    low = jnp.clip(jnp.floor(xs), 0.0, P - 2)
    feat = lax.broadcasted_iota(jnp.int32, (BATCH, IN_F), 1)
    xs_ref[...] = xs
    idx_ref[...] = low.astype(jnp.int32) + feat * P


def _prep(x):
    return pl.pallas_call(
        _prep_body,
        out_shape=(
            jax.ShapeDtypeStruct((BATCH, IN_F), jnp.float32),
            jax.ShapeDtypeStruct((BATCH, IN_F), jnp.int32),
        ),
    )(x)


_FB = 8


def _table_body(r_ref, l_ref, tb_ref):
    row = lax.broadcasted_iota(jnp.int32, (P, P), 0)
    col = lax.broadcasted_iota(jnp.int32, (P, P), 1)
    tril = jnp.where(row >= col, 1.0, 0.0).astype(jnp.float32)
    bias = lax.broadcasted_iota(jnp.int32, (P, OUT_F), 0).astype(jnp.float32)
    for i in range(_FB):
        r = r_ref[i]
        l = l_ref[i]
        u = r + l
        cat = jnp.concatenate([u, u * bias], axis=1)
        cs = jnp.dot(tril, cat, preferred_element_type=jnp.float32)
        suml = jnp.sum(l, axis=0, keepdims=True)
        sumlb = jnp.sum(l * bias, axis=0, keepdims=True)
        s_part = cs[:, :OUT_F] - suml
        i_part = sumlb - cs[:, OUT_F:]
        tb_ref[i] = jnp.concatenate([s_part, i_part], axis=1)


def _table(r_weight, l_weight):
    return pl.pallas_call(
        _table_body,
        grid=(IN_F // _FB,),
        in_specs=[
            pl.BlockSpec((_FB, P, OUT_F), lambda f: (f, 0, 0)),
            pl.BlockSpec((_FB, P, OUT_F), lambda f: (f, 0, 0)),
        ],
        out_specs=pl.BlockSpec((_FB, P, _TW), lambda f: (f, 0, 0)),
        out_shape=jax.ShapeDtypeStruct((IN_F, P, _TW), jnp.float32),
    )(r_weight, l_weight)


def _sc_body(tb_hbm, idx_hbm, xs_hbm, out_hbm,
             idx_v, xs_v, buf0, buf1, out_v, gsem0, gsem1):
    wid = lax.axis_index("s") * _NC + lax.axis_index("c")
    row0 = wid * _BPW
    bufs = (buf0, buf1)
    gsems = (gsem0, gsem1)

    def issue(bl, k):
        pltpu.make_async_copy(tb_hbm.at[idx_v.at[bl]], bufs[k], gsems[k]).start()

    def drain(k):
        pltpu.make_async_copy(tb_hbm.at[pl.ds(0, IN_F)], bufs[k], gsems[k]).wait()

    def compute(bl, k):
        buf = bufs[k]

        def gbody(fg, accs):
            a0, a1, a2, a3 = accs
            xs16 = xs_v[pl.ds(bl * IN_F + fg * 16, 16)]
            for j in range(16):
                f = fg * 16 + j
                xsv = jnp.broadcast_to(xs16[j], (16,))
                a0 = a0 + xsv * buf[f, pl.ds(0, 16)] + buf[f, pl.ds(64, 16)]
                a1 = a1 + xsv * buf[f, pl.ds(16, 16)] + buf[f, pl.ds(80, 16)]
                a2 = a2 + xsv * buf[f, pl.ds(32, 16)] + buf[f, pl.ds(96, 16)]
                a3 = a3 + xsv * buf[f, pl.ds(48, 16)] + buf[f, pl.ds(112, 16)]
            return (a0, a1, a2, a3)

        z = jnp.zeros((16,), jnp.float32)
        a0, a1, a2, a3 = lax.fori_loop(0, IN_F // 16, gbody, (z, z, z, z))
        out_v[bl, pl.ds(0, 16)] = a0
        out_v[bl, pl.ds(16, 16)] = a1
        out_v[bl, pl.ds(32, 16)] = a2
        out_v[bl, pl.ds(48, 16)] = a3

    def sb_body(s, carry):
        base = row0 + s * _SB
        pltpu.sync_copy(idx_hbm.at[pl.ds(base, _SB), :], idx_v)
        pltpu.sync_copy(xs_hbm.at[pl.ds(base * IN_F, _SB * IN_F)], xs_v)
        issue(0, 0)

        def pair_body(j, c):
            bl = 2 * j
            drain(0)
            issue(bl + 1, 1)
            compute(bl, 0)
            drain(1)

            @pl.when(bl + 2 < _SB)
            def _():
                issue(bl + 2, 0)

            compute(bl + 1, 1)
            return c

        lax.fori_loop(0, _SB // 2, pair_body, 0)
        pltpu.sync_copy(out_v, out_hbm.at[pl.ds(base, _SB), :])
        return carry

    lax.fori_loop(0, _NSB, sb_body, 0)


_sc_kan = functools.partial(
    pl.kernel,
    out_type=jax.ShapeDtypeStruct((BATCH, OUT_F), jnp.float32),
    mesh=plsc.VectorSubcoreMesh(core_axis_name="c", subcore_axis_name="s",
                                num_cores=_NC, num_subcores=_NS),
    scratch_types=[
        pltpu.VMEM((_SB, IN_F), jnp.int32),
        pltpu.VMEM((_SB * IN_F,), jnp.float32),
        pltpu.VMEM((IN_F, _TW), jnp.float32),
        pltpu.VMEM((IN_F, _TW), jnp.float32),
        pltpu.VMEM((_SB, OUT_F), jnp.float32),
        pltpu.SemaphoreType.DMA,
        pltpu.SemaphoreType.DMA,
    ],
)(_sc_body)


def kernel(x, r_weight, l_weight):
    xs, idx = _prep(x)
    tb = _table(r_weight, l_weight).reshape(IN_F * P, _TW)
    return _sc_kan(tb, idx, xs.reshape(-1))
